# Initial kernel scaffold; baseline (speedup 1.0000x reference)
import functools

import jax
import jax.numpy as jnp
from jax import lax
from jax.experimental import pallas as pl
from jax.experimental.pallas import tpu as pltpu
from jax.experimental.pallas import tpu_sc as plsc

B = 16384
D = 64
L = 16
NC = 2
NS = 16
NW = NC * NS
BPW = B // NW
IC = 128
NCHUNK = BPW // IC
NG = BPW // L
BIAS_PAD = 1024


def _predictor_body(head_hbm, tail_hbm, ids_hbm, table_hbm, bias_hbm, out_hbm,
                    idx_v, idx_flat, rel_v, head_v, tail_v, bias_v, scores_v,
                    sem):
    wid = lax.axis_index("s") * NC + lax.axis_index("c")
    base = wid * BPW
    for c in range(NCHUNK):
        pltpu.sync_copy(ids_hbm.at[pl.ds(base + c * IC, IC)], idx_v.at[c])
    copies = [
        pltpu.async_copy(table_hbm.at[idx_v.at[c]],
                         rel_v.at[pl.ds(c * IC, IC)], sem)
        for c in range(NCHUNK)
    ]
    pltpu.sync_copy(ids_hbm.at[pl.ds(base, BPW)], idx_flat)
    pltpu.sync_copy(bias_hbm, bias_v)
    pltpu.sync_copy(head_hbm.at[pl.ds(base, BPW)], head_v)
    pltpu.sync_copy(tail_hbm.at[pl.ds(base, BPW)], tail_v)
    for cp in copies:
        cp.wait()

    def group_body(g, carry):
        for j in range(L):
            b = g * L + j
            acc = ((head_v[b, pl.ds(0, L)] + rel_v[b, pl.ds(0, L)])
                   * tail_v[b, pl.ds(0, L)])
            for k in range(1, D // L):
                acc = acc + ((head_v[b, pl.ds(k * L, L)]
                              + rel_v[b, pl.ds(k * L, L)])
                             * tail_v[b, pl.ds(k * L, L)])
            scores_v[b] = jnp.sum(acc)
        idxv = idx_flat[pl.ds(g * L, L)]
        bvals = plsc.load_gather(bias_v, [idxv])
        scores_v[pl.ds(g * L, L)] = scores_v[pl.ds(g * L, L)] + bvals
        return carry

    lax.fori_loop(0, NG, group_body, 0)
    pltpu.sync_copy(scores_v, out_hbm.at[pl.ds(base, BPW)])


def kernel(head_emb, tail_emb, rel_ids, rel_emb_weight, bias):
    mesh = plsc.VectorSubcoreMesh(core_axis_name="c", subcore_axis_name="s")
    run = pl.kernel(
        _predictor_body,
        out_type=jax.ShapeDtypeStruct((B,), jnp.float32),
        mesh=mesh,
        scratch_types=[
            pltpu.VMEM((NCHUNK, IC), jnp.int32),
            pltpu.VMEM((BPW,), jnp.int32),
            pltpu.VMEM((BPW, D), jnp.float32),
            pltpu.VMEM((BPW, D), jnp.float32),
            pltpu.VMEM((BPW, D), jnp.float32),
            pltpu.VMEM((BIAS_PAD,), jnp.float32),
            pltpu.VMEM((BPW,), jnp.float32),
            pltpu.SemaphoreType.DMA,
        ],
    )
    ids32 = rel_ids.astype(jnp.int32)
    bias_p = jnp.zeros((BIAS_PAD,), jnp.float32).at[: bias.shape[0]].set(bias)
    return run(head_emb, tail_emb, ids32, rel_emb_weight, bias_p)


# speedup vs baseline: 2.1832x; 2.1832x over previous
import jax
import jax.numpy as jnp
from jax import lax
from jax.experimental import pallas as pl
from jax.experimental.pallas import tpu as pltpu
from jax.experimental.pallas import tpu_sc as plsc

B = 16384
D = 64
L = 16
NC = 2
NS = 16
NW = NC * NS
BPW = B // NW
IC = 128
NIDX = BPW // IC
CH = 256
NCH = BPW // CH
GPC = CH // IC
NG = CH // L
RPC = CH * L // IC
RCHUNK = NCH * RPC


def _predictor_body(head_hbm, tail_hbm, ids_hbm, table_hbm, bias_hbm, out_hbm,
                    idx_v, rel_v, head_v, tail_v, brow_v, part_v, ridx_flat,
                    ridx_v, shared_v, sh_idx, sem):
    s_idx = lax.axis_index("s")
    wid = s_idx * NC + lax.axis_index("c")
    base = wid * BPW
    sbase = s_idx * BPW

    for c in range(NIDX):
        pltpu.sync_copy(ids_hbm.at[pl.ds(base + c * IC, IC)], idx_v.at[c])
    bias_copies = [
        pltpu.async_copy(bias_hbm.at[idx_v.at[c]],
                         brow_v.at[pl.ds(c * IC, IC)], sem)
        for c in range(NIDX)
    ]

    def ridx_body(n, carry):
        ridx_flat[pl.ds(n * L, L)] = jnp.full((L,), n, jnp.int32) + sbase
        return carry

    lax.fori_loop(0, BPW, ridx_body, 0)
    s_off = s_idx * (RCHUNK * IC)
    pltpu.sync_copy(ridx_flat, sh_idx.at[pl.ds(s_off, RCHUNK * IC)])
    idx_rows = [
        pltpu.async_copy(sh_idx.at[pl.ds(s_off + r * IC, IC)],
                         ridx_v.at[r], sem)
        for r in range(RCHUNK)
    ]
    for cp in bias_copies + idx_rows:
        cp.wait()
    pltpu.sync_copy(brow_v, shared_v.at[pl.ds(sbase, BPW)])

    for c in range(NCH):
        row0 = c * CH
        copies = [
            pltpu.async_copy(table_hbm.at[idx_v.at[c * GPC + g]],
                             rel_v.at[pl.ds(g * IC, IC)], sem)
            for g in range(GPC)
        ]
        pltpu.sync_copy(head_hbm.at[pl.ds(base + row0, CH)], head_v)
        pltpu.sync_copy(tail_hbm.at[pl.ds(base + row0, CH)], tail_v)
        for cp in copies:
            cp.wait()

        def group_body(g, carry):
            for j in range(L):
                b = g * L + j
                acc = ((head_v[b, pl.ds(0, L)] + rel_v[b, pl.ds(0, L)])
                       * tail_v[b, pl.ds(0, L)])
                for k in range(1, D // L):
                    acc = acc + ((head_v[b, pl.ds(k * L, L)]
                                  + rel_v[b, pl.ds(k * L, L)])
                                 * tail_v[b, pl.ds(k * L, L)])
                part_v[pl.ds(b * L, L)] = acc
            return carry

        lax.fori_loop(0, NG, group_body, 0)

        adds = [
            pltpu.async_copy(part_v.at[pl.ds(r * IC, IC)],
                             shared_v.at[ridx_v.at[c * RPC + r]], sem,
                             add=True)
            for r in range(RPC)
        ]
        for cp in adds:
            cp.wait()

    pltpu.sync_copy(shared_v.at[pl.ds(sbase, BPW)],
                    out_hbm.at[pl.ds(base, BPW)])


def kernel(head_emb, tail_emb, rel_ids, rel_emb_weight, bias):
    mesh = plsc.VectorSubcoreMesh(core_axis_name="c", subcore_axis_name="s")
    run = pl.kernel(
        _predictor_body,
        out_type=jax.ShapeDtypeStruct((B,), jnp.float32),
        mesh=mesh,
        compiler_params=pltpu.CompilerParams(use_tc_tiling_on_sc=False),
        scratch_types=[
            pltpu.VMEM((NIDX, IC), jnp.int32),
            pltpu.VMEM((CH, D), jnp.float32),
            pltpu.VMEM((CH, D), jnp.float32),
            pltpu.VMEM((CH, D), jnp.float32),
            pltpu.VMEM((BPW,), jnp.float32),
            pltpu.VMEM((CH * L,), jnp.float32),
            pltpu.VMEM((RCHUNK * IC,), jnp.int32),
            pltpu.VMEM((RCHUNK, IC), jnp.int32),
            pltpu.VMEM_SHARED((NS * BPW,), jnp.float32),
            pltpu.VMEM_SHARED((NS * RCHUNK * IC,), jnp.int32),
            pltpu.SemaphoreType.DMA,
        ],
    )
    ids32 = rel_ids.astype(jnp.int32)
    return run(head_emb, tail_emb, ids32, rel_emb_weight, bias)


# speedup vs baseline: 2.2592x; 1.0348x over previous
import jax
import jax.numpy as jnp
from jax import lax
from jax.experimental import pallas as pl
from jax.experimental.pallas import tpu as pltpu
from jax.experimental.pallas import tpu_sc as plsc

B = 16384
D = 64
L = 16
NC = 2
NS = 16
NW = NC * NS
BPW = B // NW
IC = 128
NIDX = BPW // IC
CH = 256
NCH = BPW // CH
NBUF = 2
GPC = CH // IC
NG = CH // L
RPC = CH * L // IC
RCHUNK = NCH * RPC


def _predictor_body(head_hbm, tail_hbm, ids_hbm, table_hbm, bias_hbm,
                    ridx_hbm, out_hbm,
                    idx_v, rel_v, head_v, tail_v, brow_v, part_v, ridx_v,
                    shared_v, sem_misc, sem_in0, sem_in1, sem_add0, sem_add1):
    s_idx = lax.axis_index("s")
    wid = s_idx * NC + lax.axis_index("c")
    base = wid * BPW
    sbase = s_idx * BPW
    sems_in = (sem_in0, sem_in1)
    sems_add = (sem_add0, sem_add1)

    for c in range(NIDX):
        pltpu.sync_copy(ids_hbm.at[pl.ds(base + c * IC, IC)], idx_v.at[c])

    def start_inputs(c):
        p = c % NBUF
        row0 = c * CH
        return [
            pltpu.async_copy(table_hbm.at[idx_v.at[c * GPC + g]],
                             rel_v.at[p].at[pl.ds(g * IC, IC)], sems_in[p])
            for g in range(GPC)
        ] + [
            pltpu.async_copy(head_hbm.at[pl.ds(base + row0, CH)],
                             head_v.at[p], sems_in[p]),
            pltpu.async_copy(tail_hbm.at[pl.ds(base + row0, CH)],
                             tail_v.at[p], sems_in[p]),
        ]

    in_copies = [start_inputs(c) for c in range(NCH)]

    misc = [
        pltpu.async_copy(bias_hbm.at[idx_v.at[c]],
                         brow_v.at[pl.ds(c * IC, IC)], sem_misc)
        for c in range(NIDX)
    ] + [pltpu.async_copy(ridx_hbm.at[s_idx], ridx_v, sem_misc)]
    for cp in misc:
        cp.wait()
    pltpu.sync_copy(brow_v, shared_v.at[pl.ds(sbase, BPW)])

    add_copies = [None] * NCH
    for c in range(NCH):
        p = c % NBUF
        for cp in in_copies[c]:
            cp.wait()

        def group_body(g, carry):
            for j in range(L):
                b = g * L + j
                acc = ((head_v[p, b, pl.ds(0, L)] + rel_v[p, b, pl.ds(0, L)])
                       * tail_v[p, b, pl.ds(0, L)])
                for k in range(1, D // L):
                    acc = acc + ((head_v[p, b, pl.ds(k * L, L)]
                                  + rel_v[p, b, pl.ds(k * L, L)])
                                 * tail_v[p, b, pl.ds(k * L, L)])
                part_v[p, pl.ds(b * L, L)] = acc
            return carry

        lax.fori_loop(0, NG, group_body, 0)

        add_copies[c] = [
            pltpu.async_copy(part_v.at[p].at[pl.ds(r * IC, IC)],
                             shared_v.at[ridx_v.at[c * RPC + r]],
                             sems_add[p], add=True)
            for r in range(RPC)
        ]

    for c in range(NCH):
        for cp in add_copies[c]:
            cp.wait()
    pltpu.sync_copy(shared_v.at[pl.ds(sbase, BPW)],
                    out_hbm.at[pl.ds(base, BPW)])


def kernel(head_emb, tail_emb, rel_ids, rel_emb_weight, bias):
    mesh = plsc.VectorSubcoreMesh(core_axis_name="c", subcore_axis_name="s")
    run = pl.kernel(
        _predictor_body,
        out_type=jax.ShapeDtypeStruct((B,), jnp.float32),
        mesh=mesh,
        compiler_params=pltpu.CompilerParams(use_tc_tiling_on_sc=False),
        scratch_types=[
            pltpu.VMEM((NIDX, IC), jnp.int32),
            pltpu.VMEM((NBUF, CH, D), jnp.float32),
            pltpu.VMEM((NBUF, CH, D), jnp.float32),
            pltpu.VMEM((NBUF, CH, D), jnp.float32),
            pltpu.VMEM((BPW,), jnp.float32),
            pltpu.VMEM((NBUF, CH * L), jnp.float32),
            pltpu.VMEM((RCHUNK, IC), jnp.int32),
            pltpu.VMEM_SHARED((NS * BPW,), jnp.float32),
            pltpu.SemaphoreType.DMA,
            pltpu.SemaphoreType.DMA,
            pltpu.SemaphoreType.DMA,
            pltpu.SemaphoreType.DMA,
            pltpu.SemaphoreType.DMA,
        ],
    )
    ids32 = rel_ids.astype(jnp.int32)
    ridx = (jnp.arange(NS, dtype=jnp.int32)[:, None] * BPW
            + jnp.repeat(jnp.arange(BPW, dtype=jnp.int32), L)[None, :]
            ).reshape(NS, RCHUNK, IC)
    return run(head_emb, tail_emb, ids32, rel_emb_weight, bias, ridx)


# speedup vs baseline: 3.0896x; 1.3676x over previous
import jax
import jax.numpy as jnp
from jax import lax
from jax.experimental import pallas as pl
from jax.experimental.pallas import tpu as pltpu
from jax.experimental.pallas import tpu_sc as plsc

B = 16384
D = 64
DT = 128
L = 16
NC = 2
NS = 16
NW = NC * NS
BPW = B // NW
IC = 128
CH = 128
NCH = BPW // CH
NBUF = 2
NG = CH // L
RPC = CH * L // IC
RCHUNK = NCH * RPC


def _predictor_body(head_hbm, tail_hbm, ids_hbm, table_hbm, ridx_hbm, out_hbm,
                    idx_v, rel_v, head_v, tail_v, zero_v, part_v, ridx_v,
                    shared_v, sem_misc, sem_in0, sem_in1, sem_add0, sem_add1):
    s_idx = lax.axis_index("s")
    wid = s_idx * NC + lax.axis_index("c")
    base = wid * BPW
    sbase = s_idx * BPW
    sems_in = (sem_in0, sem_in1)
    sems_add = (sem_add0, sem_add1)

    for c in range(NCH):
        pltpu.sync_copy(ids_hbm.at[pl.ds(base + c * IC, IC)], idx_v.at[c])

    def start_inputs(c):
        p = c % NBUF
        row0 = c * CH
        return [
            pltpu.async_copy(table_hbm.at[idx_v.at[c]], rel_v.at[p],
                             sems_in[p]),
            pltpu.async_copy(head_hbm.at[pl.ds(base + row0, CH)],
                             head_v.at[p], sems_in[p]),
            pltpu.async_copy(tail_hbm.at[pl.ds(base + row0, CH)],
                             tail_v.at[p], sems_in[p]),
        ]

    in_copies = [None] * NCH
    for c in range(NBUF):
        in_copies[c] = start_inputs(c)

    ridx_cp = pltpu.async_copy(ridx_hbm.at[s_idx], ridx_v, sem_misc)

    def zero_body(g, carry):
        zero_v[pl.ds(g * L, L)] = jnp.zeros((L,), jnp.float32)
        return carry

    lax.fori_loop(0, BPW // L, zero_body, 0)
    ridx_cp.wait()
    pltpu.sync_copy(zero_v, shared_v.at[pl.ds(sbase, BPW)])

    add_copies = [None] * NCH
    for c in range(NCH):
        p = c % NBUF
        for cp in in_copies[c]:
            cp.wait()
        if c >= NBUF:
            for cp in add_copies[c - NBUF]:
                cp.wait()

        def group_body(g, carry):
            for j in range(L):
                b = g * L + j
                acc = ((head_v[p, b, pl.ds(0, L)] + rel_v[p, b, pl.ds(0, L)])
                       * tail_v[p, b, pl.ds(0, L)])
                for k in range(1, D // L):
                    acc = acc + ((head_v[p, b, pl.ds(k * L, L)]
                                  + rel_v[p, b, pl.ds(k * L, L)])
                                 * tail_v[p, b, pl.ds(k * L, L)])
                acc = acc + rel_v[p, b, pl.ds(D, L)]
                part_v[p, pl.ds(b * L, L)] = acc
            return carry

        lax.fori_loop(0, NG, group_body, 0)

        add_copies[c] = [
            pltpu.async_copy(part_v.at[p].at[pl.ds(r * IC, IC)],
                             shared_v.at[ridx_v.at[c * RPC + r]],
                             sems_add[p], add=True)
            for r in range(RPC)
        ]
        if c + NBUF < NCH:
            in_copies[c + NBUF] = start_inputs(c + NBUF)

    for c in range(NCH - NBUF, NCH):
        for cp in add_copies[c]:
            cp.wait()
    pltpu.sync_copy(shared_v.at[pl.ds(sbase, BPW)],
                    out_hbm.at[pl.ds(base, BPW)])


def kernel(head_emb, tail_emb, rel_ids, rel_emb_weight, bias):
    mesh = plsc.VectorSubcoreMesh(core_axis_name="c", subcore_axis_name="s")
    run = pl.kernel(
        _predictor_body,
        out_type=jax.ShapeDtypeStruct((B,), jnp.float32),
        mesh=mesh,
        scratch_types=[
            pltpu.VMEM((NCH, IC), jnp.int32),
            pltpu.VMEM((NBUF, CH, DT), jnp.float32),
            pltpu.VMEM((NBUF, CH, D), jnp.float32),
            pltpu.VMEM((NBUF, CH, D), jnp.float32),
            pltpu.VMEM((BPW,), jnp.float32),
            pltpu.VMEM((NBUF, CH * L), jnp.float32),
            pltpu.VMEM((RCHUNK, IC), jnp.int32),
            pltpu.VMEM_SHARED((NS * BPW,), jnp.float32),
            pltpu.SemaphoreType.DMA,
            pltpu.SemaphoreType.DMA,
            pltpu.SemaphoreType.DMA,
            pltpu.SemaphoreType.DMA,
            pltpu.SemaphoreType.DMA,
        ],
    )
    ids32 = rel_ids.astype(jnp.int32)
    tblx = jnp.concatenate(
        [rel_emb_weight, bias[:, None],
         jnp.zeros((rel_emb_weight.shape[0], DT - D - 1), jnp.float32)],
        axis=1)
    ridx = (jnp.arange(NS, dtype=jnp.int32)[:, None] * BPW
            + jnp.repeat(jnp.arange(BPW, dtype=jnp.int32), L)[None, :]
            ).reshape(NS, RCHUNK, IC)
    return run(head_emb, tail_emb, ids32, tblx, ridx)
